# baseline (speedup 1.0000x reference)
import functools

import jax
import jax.numpy as jnp
from jax import lax
from jax.experimental import pallas as pl
from jax.experimental.pallas import tpu as pltpu
from jax.experimental.pallas import tpu_sc as plsc

N_NODES = 10000
D = 128
E = 320000

NC = 2
NS = 16
NW = NC * NS
EPW = E // NW
NBUF = 4
N_PAD = 10240
ROWS_PER_TILE = N_PAD // NS
ZCH = 40

_mesh = plsc.VectorSubcoreMesh(core_axis_name="c", subcore_axis_name="s")


def _make_sc_agg(ch, grp, with_counts, nbuf=NBUF):
    ngrp = EPW // (grp * ch)
    assert ngrp * grp * ch == EPW

    out_type = [jax.ShapeDtypeStruct((NC, N_PAD, D), jnp.float32)]
    scratch = [
        pltpu.VMEM((grp, ch), jnp.int32),
        pltpu.VMEM((grp, ch), jnp.int32),
        [pltpu.VMEM((ch, D), jnp.float32)] * nbuf,
        pltpu.VMEM_SHARED((N_PAD, D), jnp.float32),
        [pltpu.SemaphoreType.DMA] * nbuf,
        [pltpu.SemaphoreType.DMA] * nbuf,
    ]
    if with_counts:
        assert ch == 40
        out_type.append(
            jax.ShapeDtypeStruct((10, NW, 1, 1000), jnp.float32))
        scratch.insert(3, pltpu.VMEM((N_NODES,), jnp.float32))

    def body(*refs):
        if with_counts:
            (x_hbm, src_hbm, dst_hbm, agg_out, cnt_out,
             src_v, dst_v, rows, cnt_v, acc, gsems, ssems) = refs
        else:
            (x_hbm, src_hbm, dst_hbm, agg_out,
             src_v, dst_v, rows, acc, gsems, ssems) = refs
            cnt_out = cnt_v = None

        cid = lax.axis_index("c")
        tid = lax.axis_index("s")
        wid = cid * NS + tid

        zvec = jnp.zeros((16,), jnp.float32)

        def zero_rows(r, carry):
            row = rows[0].at[r]
            for k in range(D // 16):
                row[pl.ds(k * 16, 16)] = zvec
            return carry

        lax.fori_loop(0, ZCH, zero_rows, 0)

        base = tid * ROWS_PER_TILE
        zrows = rows[0].at[pl.ds(0, ZCH)]

        def zcp(b):
            return pltpu.make_async_copy(
                zrows, acc.at[pl.ds(base + b * ZCH, ZCH)], ssems[b % nbuf])

        for b in range(ROWS_PER_TILE // ZCH):
            zcp(b).start()

        if with_counts:
            def zero_cnt(i, carry):
                cnt_v[pl.ds(i * 16, 16)] = zvec
                return carry

            lax.fori_loop(0, N_NODES // 16, zero_cnt, 0)

        for b in range(ROWS_PER_TILE // ZCH):
            zcp(b).wait()
        plsc.subcore_barrier()

        ones16 = jnp.ones((16,), jnp.float32)
        tail_mask = lax.iota(jnp.int32, 16) >= 8

        def gather(c, b):
            return pltpu.make_async_copy(x_hbm.at[src_v.at[c]], rows[b],
                                         gsems[b])

        def scat(c, b):
            return pltpu.make_async_copy(rows[b], acc.at[dst_v.at[c]],
                                         ssems[b])

        def counts(c):
            drow = dst_v.at[c]
            plsc.addupdate_scatter(cnt_v, [drow[pl.ds(0, 16)]], ones16)
            plsc.addupdate_scatter(cnt_v, [drow[pl.ds(16, 16)]], ones16)
            plsc.addupdate_scatter(cnt_v, [drow[pl.ds(24, 16)]], ones16,
                                   mask=tail_mask)

        def group(g, carry):
            pltpu.sync_copy(src_hbm.at[wid].at[g], src_v)
            pltpu.sync_copy(dst_hbm.at[wid].at[g], dst_v)
            for b in range(nbuf):
                gather(b, b).start()

            def step(t, carry2):
                for b in range(nbuf):
                    c = nbuf * t + b

                    @pl.when(c < grp)
                    def _():
                        gather(c, b).wait()
                        scat(c, b).start(add=True)
                        if with_counts:
                            counts(c)

                    @pl.when(c + nbuf < grp)
                    def _():
                        scat(c, b).wait()
                        gather(c + nbuf, b).start()

                return carry2

            lax.fori_loop(0, (grp + nbuf - 1) // nbuf, step, 0)

            for b in range(nbuf):
                c_last = [c for c in range(grp) if c % nbuf == b][-1]
                scat(c_last, b).wait()
            return carry

        lax.fori_loop(0, ngrp, group, 0)

        plsc.subcore_barrier()

        pltpu.sync_copy(acc.at[pl.ds(base, ROWS_PER_TILE)],
                        agg_out.at[cid].at[pl.ds(base, ROWS_PER_TILE)])

        if with_counts:
            for b in range(N_NODES // 1000):
                pltpu.make_async_copy(cnt_v.at[pl.ds(b * 1000, 1000)],
                                      cnt_out.at[b].at[wid].at[0],
                                      gsems[b % nbuf]).start()
            for b in range(N_NODES // 1000):
                pltpu.make_async_copy(cnt_v.at[pl.ds(b * 1000, 1000)],
                                      cnt_out.at[b].at[wid].at[0],
                                      gsems[b % nbuf]).wait()

    return pl.kernel(
        body,
        out_type=tuple(out_type) if with_counts else out_type[0],
        mesh=_mesh,
        scratch_types=scratch,
        compiler_params=pltpu.CompilerParams(needs_layout_passes=False),
    )


CH1, GRP1 = 40, 50
CH2, GRP2, NBUF2 = 80, 25, 3
_sc_agg1 = _make_sc_agg(CH1, GRP1, True)
_sc_agg2 = _make_sc_agg(CH2, GRP2, False, NBUF2)


BM = 1000


def _mm_body(x_ref, wr_ref, bl_ref, o_ref):
    o_ref[...] = lax.dot_general(
        x_ref[...], wr_ref[...], (((1,), (1,)), ((), ())),
        preferred_element_type=jnp.float32) + bl_ref[...]


def _mm(x, Wr, bl2d):
    return pl.pallas_call(
        _mm_body,
        grid=(N_NODES // BM,),
        in_specs=[
            pl.BlockSpec((BM, D), lambda m: (m, 0)),
            pl.BlockSpec((D, D), lambda m: (0, 0)),
            pl.BlockSpec((1, D), lambda m: (0, 0)),
        ],
        out_specs=pl.BlockSpec((BM, D), lambda m: (m, 0)),
        out_shape=jax.ShapeDtypeStruct((N_NODES, D), jnp.float32),
        compiler_params=pltpu.CompilerParams(
            dimension_semantics=("arbitrary",)),
    )(x, Wr, bl2d)


def _fin_body(agg_ref, cnt_ref, xr_ref, wl_ref, o_ref, *, relu):
    agg = agg_ref[0] + agg_ref[1]
    cnt = jnp.sum(cnt_ref[...], axis=(0, 1, 2))
    a = agg / jnp.maximum(cnt, 1.0)[:, None]
    acc = lax.dot_general(a, wl_ref[...], (((1,), (1,)), ((), ())),
                          preferred_element_type=jnp.float32)
    acc = acc + xr_ref[...]
    if relu:
        acc = jnp.maximum(acc, 0.0)
    o_ref[...] = acc


def _fin(agg, cnt, xr, Wl, relu):
    return pl.pallas_call(
        functools.partial(_fin_body, relu=relu),
        grid=(N_NODES // BM,),
        in_specs=[
            pl.BlockSpec((NC, BM, D), lambda m: (0, m, 0)),
            pl.BlockSpec((1, NW, 1, BM), lambda m: (m, 0, 0, 0)),
            pl.BlockSpec((BM, D), lambda m: (m, 0)),
            pl.BlockSpec((D, D), lambda m: (0, 0)),
        ],
        out_specs=pl.BlockSpec((BM, D), lambda m: (m, 0)),
        out_shape=jax.ShapeDtypeStruct((N_NODES, D), jnp.float32),
        compiler_params=pltpu.CompilerParams(
            dimension_semantics=("arbitrary",)),
    )(agg, cnt, xr, Wl)


def kernel(x, edge_index, W1l, b1l, W1r, W2l, b2l, W2r):
    src = edge_index[0].astype(jnp.int32)
    dst = edge_index[1].astype(jnp.int32)
    ngrp1 = EPW // (GRP1 * CH1)
    ngrp2 = EPW // (GRP2 * CH2)
    src1 = src.reshape(NW, ngrp1, GRP1, CH1)
    dst1 = dst.reshape(NW, ngrp1, GRP1, CH1)
    src2 = src.reshape(NW, ngrp2, GRP2, CH2)
    dst2 = dst.reshape(NW, ngrp2, GRP2, CH2)
    agg1, cnt = _sc_agg1(x, src1, dst1)
    xr1 = _mm(x, W1r, b1l.reshape(1, D))
    h = _fin(agg1, cnt, xr1, W1l, True)
    agg2 = _sc_agg2(h, src2, dst2)
    xr2 = _mm(h, W2r, b2l.reshape(1, D))
    out = _fin(agg2, cnt, xr2, W2l, False)
    return out


# speedup vs baseline: 1.0093x; 1.0093x over previous
import functools

import jax
import jax.numpy as jnp
from jax import lax
from jax.experimental import pallas as pl
from jax.experimental.pallas import tpu as pltpu
from jax.experimental.pallas import tpu_sc as plsc

N_NODES = 10000
D = 128
E = 320000

NC = 2
NS = 16
NW = NC * NS
EPW = E // NW
NBUF = 4
N_PAD = 10240
ROWS_PER_TILE = N_PAD // NS
ZCH = 40

_mesh = plsc.VectorSubcoreMesh(core_axis_name="c", subcore_axis_name="s")


def _make_sc_agg(ch, grp, with_counts, nbuf=NBUF):
    ngrp = EPW // (grp * ch)
    assert ngrp * grp * ch == EPW

    out_type = [jax.ShapeDtypeStruct((NC, N_PAD, D), jnp.float32)]
    scratch = [
        pltpu.VMEM((grp, ch), jnp.int32),
        pltpu.VMEM((grp, ch), jnp.int32),
        [pltpu.VMEM((ch, D), jnp.float32)] * nbuf,
        pltpu.VMEM_SHARED((N_PAD, D), jnp.float32),
        [pltpu.SemaphoreType.DMA] * nbuf,
        [pltpu.SemaphoreType.DMA] * nbuf,
    ]
    if with_counts:
        assert ch == 40
        out_type.append(
            jax.ShapeDtypeStruct((10, NW, 1, 1000), jnp.float32))
        scratch.insert(3, pltpu.VMEM((N_NODES,), jnp.float32))

    def body(*refs):
        if with_counts:
            (x_hbm, src_hbm, dst_hbm, agg_out, cnt_out,
             src_v, dst_v, rows, cnt_v, acc, gsems, ssems) = refs
        else:
            (x_hbm, src_hbm, dst_hbm, agg_out,
             src_v, dst_v, rows, acc, gsems, ssems) = refs
            cnt_out = cnt_v = None

        cid = lax.axis_index("c")
        tid = lax.axis_index("s")
        wid = cid * NS + tid

        zvec = jnp.zeros((16,), jnp.float32)

        def zero_rows(r, carry):
            row = rows[0].at[r]
            for k in range(D // 16):
                row[pl.ds(k * 16, 16)] = zvec
            return carry

        lax.fori_loop(0, ZCH, zero_rows, 0)

        base = tid * ROWS_PER_TILE
        zrows = rows[0].at[pl.ds(0, ZCH)]

        def zcp(b):
            return pltpu.make_async_copy(
                zrows, acc.at[pl.ds(base + b * ZCH, ZCH)], ssems[b % nbuf])

        for b in range(ROWS_PER_TILE // ZCH):
            zcp(b).start()

        if with_counts:
            def zero_cnt(i, carry):
                cnt_v[pl.ds(i * 16, 16)] = zvec
                return carry

            lax.fori_loop(0, N_NODES // 16, zero_cnt, 0)

        for b in range(ROWS_PER_TILE // ZCH):
            zcp(b).wait()
        plsc.subcore_barrier()

        ones16 = jnp.ones((16,), jnp.float32)
        tail_mask = lax.iota(jnp.int32, 16) >= 8

        def gather(c, b):
            return pltpu.make_async_copy(x_hbm.at[src_v.at[c]], rows[b],
                                         gsems[b])

        def scat(c, b):
            return pltpu.make_async_copy(rows[b], acc.at[dst_v.at[c]],
                                         ssems[b])

        def counts(c):
            drow = dst_v.at[c]
            plsc.addupdate_scatter(cnt_v, [drow[pl.ds(0, 16)]], ones16)
            plsc.addupdate_scatter(cnt_v, [drow[pl.ds(16, 16)]], ones16)
            plsc.addupdate_scatter(cnt_v, [drow[pl.ds(24, 16)]], ones16,
                                   mask=tail_mask)

        def group(g, carry):
            pltpu.sync_copy(src_hbm.at[wid].at[g], src_v)
            pltpu.sync_copy(dst_hbm.at[wid].at[g], dst_v)
            for b in range(nbuf):
                gather(b, b).start()

            def step(t, carry2):
                for b in range(nbuf):
                    c = nbuf * t + b

                    @pl.when(c < grp)
                    def _():
                        gather(c, b).wait()
                        scat(c, b).start(add=True)
                        if with_counts:
                            counts(c)

                    @pl.when(c + nbuf < grp)
                    def _():
                        scat(c, b).wait()
                        gather(c + nbuf, b).start()

                return carry2

            lax.fori_loop(0, (grp + nbuf - 1) // nbuf, step, 0)

            for b in range(nbuf):
                c_last = [c for c in range(grp) if c % nbuf == b][-1]
                scat(c_last, b).wait()
            return carry

        lax.fori_loop(0, ngrp, group, 0)

        plsc.subcore_barrier()

        pltpu.sync_copy(acc.at[pl.ds(base, ROWS_PER_TILE)],
                        agg_out.at[cid].at[pl.ds(base, ROWS_PER_TILE)])

        if with_counts:
            for b in range(N_NODES // 1000):
                pltpu.make_async_copy(cnt_v.at[pl.ds(b * 1000, 1000)],
                                      cnt_out.at[b].at[wid].at[0],
                                      gsems[b % nbuf]).start()
            for b in range(N_NODES // 1000):
                pltpu.make_async_copy(cnt_v.at[pl.ds(b * 1000, 1000)],
                                      cnt_out.at[b].at[wid].at[0],
                                      gsems[b % nbuf]).wait()

    return pl.kernel(
        body,
        out_type=tuple(out_type) if with_counts else out_type[0],
        mesh=_mesh,
        scratch_types=scratch,
        compiler_params=pltpu.CompilerParams(needs_layout_passes=False),
    )


CH1, GRP1 = 40, 50
CH2, GRP2, NBUF2 = 50, 40, 5
_sc_agg1 = _make_sc_agg(CH1, GRP1, True)
_sc_agg2 = _make_sc_agg(CH2, GRP2, False, NBUF2)


BM = 1000


def _mm_body(x_ref, wr_ref, bl_ref, o_ref):
    o_ref[...] = lax.dot_general(
        x_ref[...], wr_ref[...], (((1,), (1,)), ((), ())),
        preferred_element_type=jnp.float32) + bl_ref[...]


def _mm(x, Wr, bl2d):
    return pl.pallas_call(
        _mm_body,
        grid=(N_NODES // BM,),
        in_specs=[
            pl.BlockSpec((BM, D), lambda m: (m, 0)),
            pl.BlockSpec((D, D), lambda m: (0, 0)),
            pl.BlockSpec((1, D), lambda m: (0, 0)),
        ],
        out_specs=pl.BlockSpec((BM, D), lambda m: (m, 0)),
        out_shape=jax.ShapeDtypeStruct((N_NODES, D), jnp.float32),
        compiler_params=pltpu.CompilerParams(
            dimension_semantics=("arbitrary",)),
    )(x, Wr, bl2d)


def _fin_body(agg_ref, cnt_ref, xr_ref, wl_ref, o_ref, *, relu):
    agg = agg_ref[0] + agg_ref[1]
    cnt = jnp.sum(cnt_ref[...], axis=(0, 1, 2))
    a = agg / jnp.maximum(cnt, 1.0)[:, None]
    acc = lax.dot_general(a, wl_ref[...], (((1,), (1,)), ((), ())),
                          preferred_element_type=jnp.float32)
    acc = acc + xr_ref[...]
    if relu:
        acc = jnp.maximum(acc, 0.0)
    o_ref[...] = acc


def _fin(agg, cnt, xr, Wl, relu):
    return pl.pallas_call(
        functools.partial(_fin_body, relu=relu),
        grid=(N_NODES // BM,),
        in_specs=[
            pl.BlockSpec((NC, BM, D), lambda m: (0, m, 0)),
            pl.BlockSpec((1, NW, 1, BM), lambda m: (m, 0, 0, 0)),
            pl.BlockSpec((BM, D), lambda m: (m, 0)),
            pl.BlockSpec((D, D), lambda m: (0, 0)),
        ],
        out_specs=pl.BlockSpec((BM, D), lambda m: (m, 0)),
        out_shape=jax.ShapeDtypeStruct((N_NODES, D), jnp.float32),
        compiler_params=pltpu.CompilerParams(
            dimension_semantics=("arbitrary",)),
    )(agg, cnt, xr, Wl)


def kernel(x, edge_index, W1l, b1l, W1r, W2l, b2l, W2r):
    src = edge_index[0].astype(jnp.int32)
    dst = edge_index[1].astype(jnp.int32)
    ngrp1 = EPW // (GRP1 * CH1)
    ngrp2 = EPW // (GRP2 * CH2)
    src1 = src.reshape(NW, ngrp1, GRP1, CH1)
    dst1 = dst.reshape(NW, ngrp1, GRP1, CH1)
    src2 = src.reshape(NW, ngrp2, GRP2, CH2)
    dst2 = dst.reshape(NW, ngrp2, GRP2, CH2)
    agg1, cnt = _sc_agg1(x, src1, dst1)
    xr1 = _mm(x, W1r, b1l.reshape(1, D))
    h = _fin(agg1, cnt, xr1, W1l, True)
    agg2 = _sc_agg2(h, src2, dst2)
    xr2 = _mm(h, W2r, b2l.reshape(1, D))
    out = _fin(agg2, cnt, xr2, W2l, False)
    return out


# speedup vs baseline: 1.0096x; 1.0004x over previous
import functools

import jax
import jax.numpy as jnp
from jax import lax
from jax.experimental import pallas as pl
from jax.experimental.pallas import tpu as pltpu
from jax.experimental.pallas import tpu_sc as plsc

N_NODES = 10000
D = 128
E = 320000

NC = 2
NS = 16
NW = NC * NS
EPW = E // NW
NBUF = 4
N_PAD = 10240
ROWS_PER_TILE = N_PAD // NS
ZCH = 40

_mesh = plsc.VectorSubcoreMesh(core_axis_name="c", subcore_axis_name="s")


def _make_sc_agg(ch, grp, with_counts, nbuf=NBUF):
    ngrp = EPW // (grp * ch)
    assert ngrp * grp * ch == EPW

    out_type = [jax.ShapeDtypeStruct((NC, N_PAD, D), jnp.float32)]
    scratch = [
        pltpu.VMEM((grp, ch), jnp.int32),
        pltpu.VMEM((grp, ch), jnp.int32),
        [pltpu.VMEM((ch, D), jnp.float32)] * nbuf,
        pltpu.VMEM_SHARED((N_PAD, D), jnp.float32),
        [pltpu.SemaphoreType.DMA] * nbuf,
        [pltpu.SemaphoreType.DMA] * nbuf,
    ]
    if with_counts:
        assert ch == 40
        out_type.append(
            jax.ShapeDtypeStruct((10, NW, 1, 1000), jnp.float32))
        scratch.insert(3, pltpu.VMEM((N_NODES,), jnp.float32))

    def body(*refs):
        if with_counts:
            (x_hbm, src_hbm, dst_hbm, agg_out, cnt_out,
             src_v, dst_v, rows, cnt_v, acc, gsems, ssems) = refs
        else:
            (x_hbm, src_hbm, dst_hbm, agg_out,
             src_v, dst_v, rows, acc, gsems, ssems) = refs
            cnt_out = cnt_v = None

        cid = lax.axis_index("c")
        tid = lax.axis_index("s")
        wid = cid * NS + tid

        zvec = jnp.zeros((16,), jnp.float32)

        def zero_rows(r, carry):
            row = rows[0].at[r]
            for k in range(D // 16):
                row[pl.ds(k * 16, 16)] = zvec
            return carry

        lax.fori_loop(0, ZCH, zero_rows, 0)

        base = tid * ROWS_PER_TILE
        zrows = rows[0].at[pl.ds(0, ZCH)]

        def zcp(b):
            return pltpu.make_async_copy(
                zrows, acc.at[pl.ds(base + b * ZCH, ZCH)], ssems[b % nbuf])

        for b in range(ROWS_PER_TILE // ZCH):
            zcp(b).start()

        if with_counts:
            def zero_cnt(i, carry):
                cnt_v[pl.ds(i * 16, 16)] = zvec
                return carry

            lax.fori_loop(0, N_NODES // 16, zero_cnt, 0)

        for b in range(ROWS_PER_TILE // ZCH):
            zcp(b).wait()
        plsc.subcore_barrier()

        ones16 = jnp.ones((16,), jnp.float32)
        tail_mask = lax.iota(jnp.int32, 16) >= 8

        def gather(c, b):
            return pltpu.make_async_copy(x_hbm.at[src_v.at[c]], rows[b],
                                         gsems[b])

        def scat(c, b):
            return pltpu.make_async_copy(rows[b], acc.at[dst_v.at[c]],
                                         ssems[b])

        def counts(c):
            drow = dst_v.at[c]
            plsc.addupdate_scatter(cnt_v, [drow[pl.ds(0, 16)]], ones16)
            plsc.addupdate_scatter(cnt_v, [drow[pl.ds(16, 16)]], ones16)
            plsc.addupdate_scatter(cnt_v, [drow[pl.ds(24, 16)]], ones16,
                                   mask=tail_mask)

        def group(g, carry):
            pltpu.sync_copy(src_hbm.at[wid].at[g], src_v)
            pltpu.sync_copy(dst_hbm.at[wid].at[g], dst_v)
            for b in range(nbuf):
                gather(b, b).start()

            def step(t, carry2):
                for b in range(nbuf):
                    c = nbuf * t + b

                    @pl.when(c < grp)
                    def _():
                        gather(c, b).wait()
                        scat(c, b).start(add=True)
                        if with_counts:
                            counts(c)

                    @pl.when(c + nbuf < grp)
                    def _():
                        scat(c, b).wait()
                        gather(c + nbuf, b).start()

                return carry2

            lax.fori_loop(0, (grp + nbuf - 1) // nbuf, step, 0)

            for b in range(nbuf):
                c_last = [c for c in range(grp) if c % nbuf == b][-1]
                scat(c_last, b).wait()
            return carry

        lax.fori_loop(0, ngrp, group, 0)

        plsc.subcore_barrier()

        pltpu.sync_copy(acc.at[pl.ds(base, ROWS_PER_TILE)],
                        agg_out.at[cid].at[pl.ds(base, ROWS_PER_TILE)])

        if with_counts:
            for b in range(N_NODES // 1000):
                pltpu.make_async_copy(cnt_v.at[pl.ds(b * 1000, 1000)],
                                      cnt_out.at[b].at[wid].at[0],
                                      gsems[b % nbuf]).start()
            for b in range(N_NODES // 1000):
                pltpu.make_async_copy(cnt_v.at[pl.ds(b * 1000, 1000)],
                                      cnt_out.at[b].at[wid].at[0],
                                      gsems[b % nbuf]).wait()

    return pl.kernel(
        body,
        out_type=tuple(out_type) if with_counts else out_type[0],
        mesh=_mesh,
        scratch_types=scratch,
        compiler_params=pltpu.CompilerParams(needs_layout_passes=False),
    )


CH1, GRP1 = 40, 50
CH2, GRP2, NBUF2 = 50, 40, 5
_sc_agg1 = _make_sc_agg(CH1, GRP1, True)
_sc_agg2 = _make_sc_agg(CH2, GRP2, False, NBUF2)


BM = 1000


def _mm_body(x_ref, wr_ref, bl_ref, o_ref):
    o_ref[...] = lax.dot_general(
        x_ref[...], wr_ref[...], (((1,), (1,)), ((), ())),
        preferred_element_type=jnp.float32) + bl_ref[...]


def _mm(x, Wr, bl2d):
    return pl.pallas_call(
        _mm_body,
        grid=(N_NODES // BM,),
        in_specs=[
            pl.BlockSpec((BM, D), lambda m: (m, 0)),
            pl.BlockSpec((D, D), lambda m: (0, 0)),
            pl.BlockSpec((1, D), lambda m: (0, 0)),
        ],
        out_specs=pl.BlockSpec((BM, D), lambda m: (m, 0)),
        out_shape=jax.ShapeDtypeStruct((N_NODES, D), jnp.float32),
        compiler_params=pltpu.CompilerParams(
            dimension_semantics=("arbitrary",)),
    )(x, Wr, bl2d)


def _dense1_body(agg_ref, cnt_ref, x_ref, wl_ref, bl_ref, wr_ref,
                 w2r_ref, b2_ref, h_ref, xr2_ref):
    agg = agg_ref[0] + agg_ref[1]
    cnt = jnp.sum(cnt_ref[...], axis=(0, 1, 2))
    a = agg / jnp.maximum(cnt, 1.0)[:, None]
    acc = lax.dot_general(a, wl_ref[...], (((1,), (1,)), ((), ())),
                          preferred_element_type=jnp.float32)
    acc = acc + lax.dot_general(x_ref[...], wr_ref[...],
                                (((1,), (1,)), ((), ())),
                                preferred_element_type=jnp.float32)
    h = jnp.maximum(acc + bl_ref[...], 0.0)
    h_ref[...] = h
    xr2_ref[...] = lax.dot_general(h, w2r_ref[...], (((1,), (1,)), ((), ())),
                                   preferred_element_type=jnp.float32) + b2_ref[...]


def _dense1(agg, cnt, x, Wl, bl2d, Wr, W2r, b2_2d):
    return pl.pallas_call(
        _dense1_body,
        grid=(N_NODES // BM,),
        in_specs=[
            pl.BlockSpec((NC, BM, D), lambda m: (0, m, 0)),
            pl.BlockSpec((1, NW, 1, BM), lambda m: (m, 0, 0, 0)),
            pl.BlockSpec((BM, D), lambda m: (m, 0)),
            pl.BlockSpec((D, D), lambda m: (0, 0)),
            pl.BlockSpec((1, D), lambda m: (0, 0)),
            pl.BlockSpec((D, D), lambda m: (0, 0)),
            pl.BlockSpec((D, D), lambda m: (0, 0)),
            pl.BlockSpec((1, D), lambda m: (0, 0)),
        ],
        out_specs=[pl.BlockSpec((BM, D), lambda m: (m, 0)),
                   pl.BlockSpec((BM, D), lambda m: (m, 0))],
        out_shape=[jax.ShapeDtypeStruct((N_NODES, D), jnp.float32),
                   jax.ShapeDtypeStruct((N_NODES, D), jnp.float32)],
        compiler_params=pltpu.CompilerParams(
            dimension_semantics=("arbitrary",)),
    )(agg, cnt, x, Wl, bl2d, Wr, W2r, b2_2d)


def _fin_body(agg_ref, cnt_ref, xr_ref, wl_ref, o_ref, *, relu):
    agg = agg_ref[0] + agg_ref[1]
    cnt = jnp.sum(cnt_ref[...], axis=(0, 1, 2))
    a = agg / jnp.maximum(cnt, 1.0)[:, None]
    acc = lax.dot_general(a, wl_ref[...], (((1,), (1,)), ((), ())),
                          preferred_element_type=jnp.float32)
    acc = acc + xr_ref[...]
    if relu:
        acc = jnp.maximum(acc, 0.0)
    o_ref[...] = acc


def _fin(agg, cnt, xr, Wl, relu):
    return pl.pallas_call(
        functools.partial(_fin_body, relu=relu),
        grid=(N_NODES // BM,),
        in_specs=[
            pl.BlockSpec((NC, BM, D), lambda m: (0, m, 0)),
            pl.BlockSpec((1, NW, 1, BM), lambda m: (m, 0, 0, 0)),
            pl.BlockSpec((BM, D), lambda m: (m, 0)),
            pl.BlockSpec((D, D), lambda m: (0, 0)),
        ],
        out_specs=pl.BlockSpec((BM, D), lambda m: (m, 0)),
        out_shape=jax.ShapeDtypeStruct((N_NODES, D), jnp.float32),
        compiler_params=pltpu.CompilerParams(
            dimension_semantics=("arbitrary",)),
    )(agg, cnt, xr, Wl)


def kernel(x, edge_index, W1l, b1l, W1r, W2l, b2l, W2r):
    src = edge_index[0].astype(jnp.int32)
    dst = edge_index[1].astype(jnp.int32)
    ngrp1 = EPW // (GRP1 * CH1)
    ngrp2 = EPW // (GRP2 * CH2)
    src1 = src.reshape(NW, ngrp1, GRP1, CH1)
    dst1 = dst.reshape(NW, ngrp1, GRP1, CH1)
    src2 = src.reshape(NW, ngrp2, GRP2, CH2)
    dst2 = dst.reshape(NW, ngrp2, GRP2, CH2)
    agg1, cnt = _sc_agg1(x, src1, dst1)
    h, xr2 = _dense1(agg1, cnt, x, W1l, b1l.reshape(1, D), W1r,
                     W2r, b2l.reshape(1, D))
    agg2 = _sc_agg2(h, src2, dst2)
    out = _fin(agg2, cnt, xr2, W2l, False)
    return out
